# baseline (speedup 1.0000x reference)
import functools

import jax
import jax.numpy as jnp
from jax import lax
from jax.experimental import pallas as pl
from jax.experimental.pallas import tpu as pltpu
from jax.experimental.pallas import tpu_sc as plsc

T = 2048
D = 1024
E = 8
EP = 128
M = 128
NT = (2 * T) // M + E
NP = NT * M
MMG = 8

NC, NS = 2, 16
NW = NC * NS
TPW = T // NW


def _router_body(h_ref, gw_ref, gb_ref, dst1_ref, dst2_ref, te_ref,
                 bflag_ref, slot_ref, nre_ref, dopre_ref):
    h = h_ref[...]
    logits8 = jnp.dot(h, gw_ref[...], preferred_element_type=jnp.float32)
    logits8 = logits8 + gb_ref[...]
    logits = jnp.concatenate(
        [logits8, jnp.full((T, EP - E), -1e30, jnp.float32)], axis=1)
    mx = jnp.max(logits, axis=1, keepdims=True)
    un = jnp.exp(logits - mx)
    p = un / jnp.sum(un, axis=1, keepdims=True)

    iota_e = lax.broadcasted_iota(jnp.int32, (T, EP), 1)
    m1 = jnp.max(p, axis=1, keepdims=True)
    e1 = jnp.min(jnp.where(p == m1, iota_e, EP), axis=1, keepdims=True)
    p2 = jnp.where(iota_e == e1, -jnp.inf, p)
    m2 = jnp.max(p2, axis=1, keepdims=True)
    e2 = jnp.min(jnp.where(p2 == m2, iota_e, EP), axis=1, keepdims=True)

    oh1 = (iota_e == e1).astype(jnp.float32)
    oh2 = (iota_e == e2).astype(jnp.float32)

    def excl_cumsum_rows(x):
        acc = x
        s = 1
        while s < T:
            acc = acc + jnp.concatenate(
                [jnp.zeros((s, EP), jnp.float32), acc[: T - s, :]], axis=0)
            s *= 2
        return acc - x

    r1 = excl_cumsum_rows(oh1)
    r2 = excl_cumsum_rows(oh2)
    c1 = jnp.sum(oh1, axis=0, keepdims=True)
    c2 = jnp.sum(oh2, axis=0, keepdims=True)
    counts = c1 + c2
    pc = jnp.ceil(counts / M) * M

    incl = pc
    s = 1
    while s < EP:
        incl = incl + jnp.concatenate(
            [jnp.zeros((1, s), jnp.float32), incl[:, : EP - s]], axis=1)
        s *= 2
    off = incl - pc

    d1 = jnp.sum((r1 + off) * oh1, axis=1, keepdims=True)
    d2 = jnp.sum((r2 + off + c1) * oh2, axis=1, keepdims=True)
    dst1_ref[...] = d1.astype(jnp.int32)
    dst2_ref[...] = d2.astype(jnp.int32)

    starts = (lax.broadcasted_iota(jnp.int32, (NT, EP), 0) * M).astype(
        jnp.float32)
    te = jnp.sum((starts >= incl).astype(jnp.int32), axis=1, keepdims=True)
    present0 = (pc > 0.5)
    lane0 = lax.broadcasted_iota(jnp.int32, (NT, EP), 1)
    lp = jnp.max(jnp.where(present0, lane0, 0), axis=1, keepdims=True)
    te = jnp.minimum(te, lp)
    te_ref[...] = te

    te_prev = jnp.concatenate([jnp.full((1, 1), -1, jnp.int32), te[:-1, :]],
                              axis=0)
    bflag = (te != te_prev).astype(jnp.int32)
    runidx = bflag
    s = 1
    while s < NT:
        runidx = runidx + jnp.concatenate(
            [jnp.zeros((s, 1), jnp.int32), runidx[: NT - s, :]], axis=0)
        s *= 2
    slot = (runidx - 1) % 2
    cand = jnp.where((lane0 > te) & present0, lane0, EP)
    nre = jnp.min(cand, axis=1, keepdims=True)
    dopre = ((bflag == 1) & (nre < E)).astype(jnp.int32)
    bflag_ref[...] = bflag
    slot_ref[...] = slot
    nre_ref[...] = jnp.minimum(nre, E - 1)
    dopre_ref[...] = dopre


def _router(h, gw_p, gb_p):
    return pl.pallas_call(
        _router_body,
        out_shape=(
            jax.ShapeDtypeStruct((T, 1), jnp.int32),
            jax.ShapeDtypeStruct((T, 1), jnp.int32),
            jax.ShapeDtypeStruct((NT, 1), jnp.int32),
            jax.ShapeDtypeStruct((NT, 1), jnp.int32),
            jax.ShapeDtypeStruct((NT, 1), jnp.int32),
            jax.ShapeDtypeStruct((NT, 1), jnp.int32),
            jax.ShapeDtypeStruct((NT, 1), jnp.int32),
        ),
    )(h, gw_p, gb_p)


@functools.cache
def _get_dispatch():
    mesh = plsc.VectorSubcoreMesh(core_axis_name="c", subcore_axis_name="s")

    @functools.partial(
        pl.kernel,
        mesh=mesh,
        out_type=jax.ShapeDtypeStruct((NP, D), jnp.float32),
        scratch_types=[
            pltpu.VMEM((TPW,), jnp.int32),
            pltpu.VMEM((TPW,), jnp.int32),
            pltpu.VMEM((TPW, D), jnp.float32),
            pltpu.SemaphoreType.DMA,
            pltpu.SemaphoreType.DMA,
        ],
    )
    def _dispatch(h_hbm, d1_hbm, d2_hbm, hs_hbm, i1_v, i2_v, rows_v,
                  sem1, sem2):
        wid = lax.axis_index("s") * NC + lax.axis_index("c")
        base = wid * TPW
        pltpu.sync_copy(d1_hbm.at[pl.ds(base, TPW)], i1_v)
        pltpu.sync_copy(d2_hbm.at[pl.ds(base, TPW)], i2_v)
        pltpu.sync_copy(h_hbm.at[pl.ds(base, TPW)], rows_v)
        c1 = pltpu.async_copy(rows_v, hs_hbm.at[i1_v], sem1)
        c2 = pltpu.async_copy(rows_v, hs_hbm.at[i2_v], sem2)
        c1.wait()
        c2.wait()

    return _dispatch


def _mm_body(te_ref, bflag_ref, slot_ref, nre_ref, dopre_ref,
             hs_ref, w_hbm, ys_ref, wbuf, wsem):
    def _wcopy(expert, slot):
        return pltpu.make_async_copy(
            w_hbm.at[expert], wbuf.at[slot], wsem.at[slot])

    def tile_body(i, carry):
        e = te_ref[i, 0]
        s = slot_ref[i, 0]

        @pl.when(bflag_ref[i, 0] == 1)
        def _run_start():
            @pl.when(i == 0)
            def _first():
                _wcopy(e, s).start()

            _wcopy(e, s).wait()

            @pl.when(dopre_ref[i, 0] == 1)
            def _prefetch():
                _wcopy(nre_ref[i, 0], 1 - s).start()

        w_bf = wbuf[s].astype(jnp.bfloat16)
        j = i % (NT // MMG)
        hs_bf = hs_ref[pl.ds(j * M, M), :].astype(jnp.bfloat16)
        ys_ref[pl.ds(j * M, M), :] = jnp.dot(
            hs_bf, w_bf, preferred_element_type=jnp.float32)
        return carry

    p = pl.program_id(0)
    tpg = NT // MMG
    lax.fori_loop(p * tpg, (p + 1) * tpg, tile_body, 0)


def _expert_mm(te, bflag, slot, nre, dopre, hs, expert_w):
    grid_spec = pltpu.PrefetchScalarGridSpec(
        num_scalar_prefetch=5,
        grid=(MMG,),
        in_specs=[
            pl.BlockSpec((NP // MMG, D), lambda p, *_: (p, 0)),
            pl.BlockSpec(memory_space=pl.ANY),
        ],
        out_specs=pl.BlockSpec((NP // MMG, D), lambda p, *_: (p, 0)),
        scratch_shapes=[
            pltpu.VMEM((2, D, D), jnp.float32),
            pltpu.SemaphoreType.DMA((2,)),
        ],
    )
    return pl.pallas_call(
        _mm_body,
        grid_spec=grid_spec,
        out_shape=jax.ShapeDtypeStruct((NP, D), jnp.float32),
    )(te, bflag, slot, nre, dopre, hs, expert_w)


@functools.cache
def _get_combine():
    mesh = plsc.VectorSubcoreMesh(core_axis_name="c", subcore_axis_name="s")

    CH = 32

    @functools.partial(
        pl.kernel,
        mesh=mesh,
        out_type=jax.ShapeDtypeStruct((T, D), jnp.float32),
        scratch_types=[
            pltpu.VMEM((CH,), jnp.int32),
            pltpu.VMEM((CH,), jnp.int32),
            pltpu.VMEM((CH, D), jnp.float32),
            pltpu.VMEM((CH, D), jnp.float32),
            pltpu.SemaphoreType.DMA,
            pltpu.SemaphoreType.DMA,
        ],
    )
    def _combine(ys_hbm, d1_hbm, d2_hbm, out_hbm, i1_v, i2_v, b1_v, b2_v,
                 sem1, sem2):
        wid = lax.axis_index("s") * NC + lax.axis_index("c")
        for c in range(TPW // CH):
            base = wid * TPW + c * CH
            pltpu.sync_copy(d1_hbm.at[pl.ds(base, CH)], i1_v)
            pltpu.sync_copy(d2_hbm.at[pl.ds(base, CH)], i2_v)
            g1 = pltpu.async_copy(ys_hbm.at[i1_v], b1_v, sem1)
            g2 = pltpu.async_copy(ys_hbm.at[i2_v], b2_v, sem2)
            g1.wait()
            g2.wait()

            def row_body(r, _):
                for j in range(D // 16):
                    sl = pl.ds(j * 16, 16)
                    b1_v[r, sl] = b1_v[r, sl] + b2_v[r, sl]
                return 0

            lax.fori_loop(0, CH, row_body, 0)
            pltpu.sync_copy(b1_v, out_hbm.at[pl.ds(base, CH)])

    return _combine


@jax.jit
def kernel(E_symb, gate_w, gate_b, expert_w):
    h = E_symb.reshape(T, D)
    dst1, dst2, te, bflag, slot, nre, dopre = _router(
        h, gate_w, gate_b.reshape(1, E))
    dst1 = dst1.reshape(T)
    dst2 = dst2.reshape(T)

    hs = _get_dispatch()(h, dst1, dst2)
    ys = _expert_mm(te, bflag, slot, nre, dopre, hs, expert_w)
    out = _get_combine()(ys, dst1, dst2)
    return out.reshape(E_symb.shape)


# speedup vs baseline: 1.0577x; 1.0577x over previous
import functools

import jax
import jax.numpy as jnp
from jax import lax
from jax.experimental import pallas as pl
from jax.experimental.pallas import tpu as pltpu
from jax.experimental.pallas import tpu_sc as plsc

T = 2048
D = 1024
E = 8
EP = 128
M = 128
NT = (2 * T) // M + E
NP = NT * M
MMG = 8

NC, NS = 2, 16
NW = NC * NS
TPW = T // NW


def _router_body(h_ref, gw_ref, gb_ref, dst1_ref, dst2_ref, te_ref,
                 bflag_ref, slot_ref, nre_ref, dopre_ref):
    h = h_ref[...]
    logits8 = jnp.dot(h, gw_ref[...], preferred_element_type=jnp.float32)
    logits8 = logits8 + gb_ref[...]
    logits = jnp.concatenate(
        [logits8, jnp.full((T, EP - E), -1e30, jnp.float32)], axis=1)
    mx = jnp.max(logits, axis=1, keepdims=True)
    un = jnp.exp(logits - mx)
    p = un / jnp.sum(un, axis=1, keepdims=True)

    iota_e = lax.broadcasted_iota(jnp.int32, (T, EP), 1)
    m1 = jnp.max(p, axis=1, keepdims=True)
    e1 = jnp.min(jnp.where(p == m1, iota_e, EP), axis=1, keepdims=True)
    p2 = jnp.where(iota_e == e1, -jnp.inf, p)
    m2 = jnp.max(p2, axis=1, keepdims=True)
    e2 = jnp.min(jnp.where(p2 == m2, iota_e, EP), axis=1, keepdims=True)

    oh1 = (iota_e == e1).astype(jnp.float32)
    oh2 = (iota_e == e2).astype(jnp.float32)

    def excl_cumsum_rows(x):
        acc = x
        s = 1
        while s < T:
            acc = acc + jnp.concatenate(
                [jnp.zeros((s, EP), jnp.float32), acc[: T - s, :]], axis=0)
            s *= 2
        return acc - x

    r1 = excl_cumsum_rows(oh1)
    r2 = excl_cumsum_rows(oh2)
    c1 = jnp.sum(oh1, axis=0, keepdims=True)
    c2 = jnp.sum(oh2, axis=0, keepdims=True)
    counts = c1 + c2
    pc = jnp.ceil(counts / M) * M

    incl = pc
    s = 1
    while s < EP:
        incl = incl + jnp.concatenate(
            [jnp.zeros((1, s), jnp.float32), incl[:, : EP - s]], axis=1)
        s *= 2
    off = incl - pc

    d1 = jnp.sum((r1 + off) * oh1, axis=1, keepdims=True)
    d2 = jnp.sum((r2 + off + c1) * oh2, axis=1, keepdims=True)
    dst1_ref[...] = d1.astype(jnp.int32).reshape(T // 128, 128)
    dst2_ref[...] = d2.astype(jnp.int32).reshape(T // 128, 128)

    starts = (lax.broadcasted_iota(jnp.int32, (NT, EP), 0) * M).astype(
        jnp.float32)
    te = jnp.sum((starts >= incl).astype(jnp.int32), axis=1, keepdims=True)
    present0 = (pc > 0.5)
    lane0 = lax.broadcasted_iota(jnp.int32, (NT, EP), 1)
    lp = jnp.max(jnp.where(present0, lane0, 0), axis=1, keepdims=True)
    te = jnp.minimum(te, lp)
    te_ref[...] = te

    te_prev = jnp.concatenate([jnp.full((1, 1), -1, jnp.int32), te[:-1, :]],
                              axis=0)
    bflag = (te != te_prev).astype(jnp.int32)
    runidx = bflag
    s = 1
    while s < NT:
        runidx = runidx + jnp.concatenate(
            [jnp.zeros((s, 1), jnp.int32), runidx[: NT - s, :]], axis=0)
        s *= 2
    slot = (runidx - 1) % 2
    cand = jnp.where((lane0 > te) & present0, lane0, EP)
    nre = jnp.min(cand, axis=1, keepdims=True)
    dopre = ((bflag == 1) & (nre < E)).astype(jnp.int32)
    bflag_ref[...] = bflag
    slot_ref[...] = slot
    nre_ref[...] = jnp.minimum(nre, E - 1)
    dopre_ref[...] = dopre


def _router(h, gw_p, gb_p):
    return pl.pallas_call(
        _router_body,
        out_shape=(
            jax.ShapeDtypeStruct((T // 128, 128), jnp.int32),
            jax.ShapeDtypeStruct((T // 128, 128), jnp.int32),
            jax.ShapeDtypeStruct((NT, 1), jnp.int32),
            jax.ShapeDtypeStruct((NT, 1), jnp.int32),
            jax.ShapeDtypeStruct((NT, 1), jnp.int32),
            jax.ShapeDtypeStruct((NT, 1), jnp.int32),
            jax.ShapeDtypeStruct((NT, 1), jnp.int32),
        ),
    )(h, gw_p, gb_p)


@functools.cache
def _get_dispatch():
    mesh = plsc.VectorSubcoreMesh(core_axis_name="c", subcore_axis_name="s")

    @functools.partial(
        pl.kernel,
        mesh=mesh,
        out_type=jax.ShapeDtypeStruct((NP, D), jnp.float32),
        scratch_types=[
            pltpu.VMEM((TPW,), jnp.int32),
            pltpu.VMEM((TPW,), jnp.int32),
            pltpu.VMEM((TPW, D), jnp.float32),
            pltpu.SemaphoreType.DMA,
            pltpu.SemaphoreType.DMA,
        ],
    )
    def _dispatch(h_hbm, d1_hbm, d2_hbm, hs_hbm, i1_v, i2_v, rows_v,
                  sem1, sem2):
        wid = lax.axis_index("s") * NC + lax.axis_index("c")
        base = wid * TPW
        pltpu.sync_copy(d1_hbm.at[pl.ds(base, TPW)], i1_v)
        pltpu.sync_copy(d2_hbm.at[pl.ds(base, TPW)], i2_v)
        pltpu.sync_copy(h_hbm.at[pl.ds(base, TPW)], rows_v)
        c1 = pltpu.async_copy(rows_v, hs_hbm.at[i1_v], sem1)
        c2 = pltpu.async_copy(rows_v, hs_hbm.at[i2_v], sem2)
        c1.wait()
        c2.wait()

    return _dispatch


def _mm_body(te_ref, bflag_ref, slot_ref, nre_ref, dopre_ref,
             hs_ref, w_hbm, ys_ref, wbuf, wsem):
    def _wcopy(expert, slot):
        return pltpu.make_async_copy(
            w_hbm.at[expert], wbuf.at[slot], wsem.at[slot])

    def tile_body(i, carry):
        e = te_ref[i, 0]
        s = slot_ref[i, 0]

        @pl.when(bflag_ref[i, 0] == 1)
        def _run_start():
            @pl.when(i == 0)
            def _first():
                _wcopy(e, s).start()

            _wcopy(e, s).wait()

            @pl.when(dopre_ref[i, 0] == 1)
            def _prefetch():
                _wcopy(nre_ref[i, 0], 1 - s).start()

        w_bf = wbuf[s].astype(jnp.bfloat16)
        j = i % (NT // MMG)
        hs_bf = hs_ref[pl.ds(j * M, M), :].astype(jnp.bfloat16)
        ys_ref[pl.ds(j * M, M), :] = jnp.dot(
            hs_bf, w_bf, preferred_element_type=jnp.float32)
        return carry

    p = pl.program_id(0)
    tpg = NT // MMG
    lax.fori_loop(p * tpg, (p + 1) * tpg, tile_body, 0)


def _expert_mm(te, bflag, slot, nre, dopre, hs, expert_w):
    grid_spec = pltpu.PrefetchScalarGridSpec(
        num_scalar_prefetch=5,
        grid=(MMG,),
        in_specs=[
            pl.BlockSpec((NP // MMG, D), lambda p, *_: (p, 0)),
            pl.BlockSpec(memory_space=pl.ANY),
        ],
        out_specs=pl.BlockSpec((NP // MMG, D), lambda p, *_: (p, 0)),
        scratch_shapes=[
            pltpu.VMEM((2, D, D), jnp.float32),
            pltpu.SemaphoreType.DMA((2,)),
        ],
    )
    return pl.pallas_call(
        _mm_body,
        grid_spec=grid_spec,
        out_shape=jax.ShapeDtypeStruct((NP, D), jnp.float32),
    )(te, bflag, slot, nre, dopre, hs, expert_w)


@functools.cache
def _get_combine():
    mesh = plsc.VectorSubcoreMesh(core_axis_name="c", subcore_axis_name="s")

    CH = 32

    @functools.partial(
        pl.kernel,
        mesh=mesh,
        out_type=jax.ShapeDtypeStruct((T, D), jnp.float32),
        scratch_types=[
            pltpu.VMEM((CH,), jnp.int32),
            pltpu.VMEM((CH,), jnp.int32),
            pltpu.VMEM((CH, D), jnp.float32),
            pltpu.VMEM((CH, D), jnp.float32),
            pltpu.SemaphoreType.DMA,
            pltpu.SemaphoreType.DMA,
        ],
    )
    def _combine(ys_hbm, d1_hbm, d2_hbm, out_hbm, i1_v, i2_v, b1_v, b2_v,
                 sem1, sem2):
        wid = lax.axis_index("s") * NC + lax.axis_index("c")
        for c in range(TPW // CH):
            base = wid * TPW + c * CH
            pltpu.sync_copy(d1_hbm.at[pl.ds(base, CH)], i1_v)
            pltpu.sync_copy(d2_hbm.at[pl.ds(base, CH)], i2_v)
            g1 = pltpu.async_copy(ys_hbm.at[i1_v], b1_v, sem1)
            g2 = pltpu.async_copy(ys_hbm.at[i2_v], b2_v, sem2)
            g1.wait()
            g2.wait()

            def row_body(r, _):
                for j in range(D // 16):
                    sl = pl.ds(j * 16, 16)
                    b1_v[r, sl] = b1_v[r, sl] + b2_v[r, sl]
                return 0

            lax.fori_loop(0, CH, row_body, 0)
            pltpu.sync_copy(b1_v, out_hbm.at[pl.ds(base, CH)])

    return _combine


@jax.jit
def kernel(E_symb, gate_w, gate_b, expert_w):
    h = E_symb.reshape(T, D)
    dst1, dst2, te, bflag, slot, nre, dopre = _router(
        h, gate_w, gate_b.reshape(1, E))
    dst1 = dst1.reshape(T)
    dst2 = dst2.reshape(T)

    hs = _get_dispatch()(h, dst1, dst2)
    ys = _expert_mm(te, bflag, slot, nre, dopre, hs, expert_w)
    out = _get_combine()(ys, dst1, dst2)
    return out.reshape(E_symb.shape)
